# baseline (speedup 1.0000x reference)
import functools

import jax
import jax.numpy as jnp
from jax import lax
from jax.experimental import pallas as pl
from jax.experimental.pallas import tpu as pltpu
from jax.experimental.pallas import tpu_sc as plsc

VOCAB = 1000000
DIM = 64
B = 4096
L = 200

NUM_CORES = 2
NUM_SUBCORES = 16
NW = NUM_CORES * NUM_SUBCORES
B_PER_W = B // NW
IDX_PER_W = B_PER_W * L
CH = 40
LANES = 16
NCH = DIM // LANES

_mesh = plsc.VectorSubcoreMesh(
    core_axis_name="c", subcore_axis_name="s",
    num_cores=NUM_CORES, num_subcores=NUM_SUBCORES)


@functools.partial(
    pl.kernel,
    out_type=jax.ShapeDtypeStruct((B * DIM,), jnp.float32),
    mesh=_mesh,
    scratch_types=[
        pltpu.VMEM((IDX_PER_W,), jnp.int32),
        pltpu.VMEM((IDX_PER_W,), jnp.int32),
        pltpu.VMEM((L, 2 * DIM), jnp.float32),
        pltpu.VMEM((L, 2 * DIM), jnp.float32),
        pltpu.VMEM((B_PER_W * DIM,), jnp.float32),
        pltpu.SemaphoreType.DMA,
        pltpu.SemaphoreType.DMA,
    ],
    compiler_params=pltpu.CompilerParams(use_tc_tiling_on_sc=True),
)
def _embed_mean(q_hbm, p_hbm, table_hbm, out_hbm, idx_v, par_v, rows0, rows1,
                out_v, sem0, sem1):
    wid = lax.axis_index("s") * NUM_CORES + lax.axis_index("c")

    pltpu.sync_copy(q_hbm.at[pl.ds(wid * IDX_PER_W, IDX_PER_W)], idx_v)
    pltpu.sync_copy(p_hbm.at[pl.ds(wid * IDX_PER_W, IDX_PER_W)], par_v)

    def issue(buf, sem, b):
        for j in range(L // LANES):
            vec = idx_v[pl.ds(b * L + j * LANES, LANES)]
            pltpu.async_copy(table_hbm.at[vec],
                             buf.at[pl.ds(j * LANES, LANES)], sem)
        tail = (L // LANES) * LANES
        pltpu.async_copy(
            table_hbm.at[idx_v.at[pl.ds(b * L + tail, L - tail)]],
            buf.at[pl.ds(tail, L - tail)], sem)

    def wait(buf, sem):
        pltpu.make_async_copy(table_hbm.at[pl.ds(0, L)], buf, sem).wait()

    ring = ((rows0, sem0, 0), (rows1, sem1, 1))
    NBUF = len(ring)
    for buf, sem, off in ring:
        issue(buf, sem, off)

    scale = jnp.float32(1.0 / L)
    UNROLL = 8

    def accumulate(buf, b):
        def acc_body(i, carry):
            acc = list(carry)
            base = i * UNROLL
            for r in range(UNROLL):
                row = base + r
                pvec = par_v[pl.ds(0, LANES)]
                mask = pvec > 0
                for c in range(NCH):
                    lo = buf[row, pl.ds(c * LANES, LANES)]
                    hi = buf[row, pl.ds(DIM + c * LANES, LANES)]
                    k = c * 2 + (r % 2)
                    acc[k] = acc[k] + jnp.where(mask, hi, lo)
            return tuple(acc)

        zero = jnp.zeros((LANES,), jnp.float32)
        acc = lax.fori_loop(0, L // UNROLL, acc_body, (zero,) * (2 * NCH))
        for c in range(NCH):
            out_v[pl.ds(b * DIM + c * LANES, LANES)] = (
                acc[c * 2] + acc[c * 2 + 1]) * scale

    def outer_body(i, carry):
        b0 = NBUF * i
        for buf, sem, off in ring:
            b = b0 + off
            wait(buf, sem)
            accumulate(buf, b)

            @pl.when(b + NBUF < B_PER_W)
            def _():
                issue(buf, sem, b + NBUF)
        return carry

    lax.fori_loop(0, B_PER_W // NBUF, outer_body, 0)

    pltpu.sync_copy(out_v, out_hbm.at[pl.ds(wid * B_PER_W * DIM,
                                            B_PER_W * DIM)])


def kernel(indices, table):
    flat = indices.reshape(-1)
    q = lax.shift_right_logical(flat, 1)
    p = jnp.bitwise_and(flat, 1)
    table2 = table.reshape(VOCAB // 2, 2 * DIM)
    out = _embed_mean(q, p, table2)
    return out.reshape(B, DIM)


# speedup vs baseline: 1.1627x; 1.1627x over previous
import functools

import jax
import jax.numpy as jnp
from jax import lax
from jax.experimental import pallas as pl
from jax.experimental.pallas import tpu as pltpu
from jax.experimental.pallas import tpu_sc as plsc

VOCAB = 1000000
DIM = 64
B = 4096
L = 200

NUM_CORES = 2
NUM_SUBCORES = 16
NW = NUM_CORES * NUM_SUBCORES
B_PER_W = B // NW
IDX_PER_W = B_PER_W * L
BPG = 2
GROWS = BPG * L
NGRP = B_PER_W // BPG
NBUF = 4
NISLOT = 2 * NBUF
LANES = 16
NCH = DIM // LANES
OUT_PER_W = B_PER_W * DIM

_mesh = plsc.VectorSubcoreMesh(
    core_axis_name="c", subcore_axis_name="s",
    num_cores=NUM_CORES, num_subcores=NUM_SUBCORES)


@functools.partial(
    pl.kernel,
    out_type=jax.ShapeDtypeStruct((B * DIM,), jnp.float32),
    mesh=_mesh,
    scratch_types=[
        [pltpu.VMEM((GROWS,), jnp.int32) for _ in range(NISLOT)],
        [pltpu.VMEM((GROWS, DIM), jnp.float32) for _ in range(NBUF)],
        pltpu.VMEM((OUT_PER_W,), jnp.float32),
        [pltpu.SemaphoreType.DMA for _ in range(NISLOT)],
        [pltpu.SemaphoreType.DMA for _ in range(NBUF)],
        pltpu.SemaphoreType.DMA,
    ],
    compiler_params=pltpu.CompilerParams(use_tc_tiling_on_sc=False),
)
def _embed_mean(idx_hbm, table_hbm, out_hbm, islots, bufs, out_v, isems,
                rsems, osem):
    wid = lax.axis_index("s") * NUM_CORES + lax.axis_index("c")
    ibase = wid * IDX_PER_W

    def stage_idx(t, g):
        pltpu.async_copy(idx_hbm.at[pl.ds(ibase + g * GROWS, GROWS)],
                         islots[t], isems[t])

    def wait_idx(t):
        pltpu.make_async_copy(idx_hbm.at[pl.ds(0, GROWS)], islots[t],
                              isems[t]).wait()

    def gather(s, t):
        pltpu.async_copy(table_hbm.at[islots[t]], bufs[s], rsems[s])

    def wait_rows(s):
        pltpu.make_async_copy(table_hbm.at[pl.ds(0, GROWS)], bufs[s],
                              rsems[s]).wait()

    for t in range(NISLOT):
        stage_idx(t, t)
    for s in range(NBUF):
        wait_idx(s)
        gather(s, s)

    scale = jnp.float32(1.0 / L)
    UNROLL = 8

    def accumulate(buf, g):
        for j in range(BPG):
            b = g * BPG + j

            def acc_body(i, carry, j=j):
                acc = list(carry)
                base = j * L + i * UNROLL
                for r in range(UNROLL):
                    for c in range(NCH):
                        k = c * 2 + (r % 2)
                        acc[k] = acc[k] + buf[base + r,
                                              pl.ds(c * LANES, LANES)]
                return tuple(acc)

            zero = jnp.zeros((LANES,), jnp.float32)
            acc = lax.fori_loop(0, L // UNROLL, acc_body, (zero,) * (2 * NCH))
            for c in range(NCH):
                out_v[pl.ds(b * DIM + c * LANES, LANES)] = (
                    acc[c * 2] + acc[c * 2 + 1]) * scale

    def body(i, carry):
        g0 = NISLOT * i
        for off in range(NISLOT):
            g = g0 + off
            s = off % NBUF
            wait_rows(s)
            accumulate(bufs[s], g)

            @pl.when(g + NBUF < NGRP)
            def _():
                wait_idx((off + NBUF) % NISLOT)
                gather(s, (off + NBUF) % NISLOT)

            @pl.when(g + NISLOT < NGRP)
            def _():
                stage_idx(off, g + NISLOT)
        return carry

    HALF_ITERS = NGRP // NISLOT // 2
    lax.fori_loop(0, HALF_ITERS, body, 0)
    pltpu.async_copy(out_v.at[pl.ds(0, OUT_PER_W // 2)],
                     out_hbm.at[pl.ds(wid * OUT_PER_W, OUT_PER_W // 2)],
                     osem)
    lax.fori_loop(HALF_ITERS, NGRP // NISLOT, body, 0)
    pltpu.async_copy(
        out_v.at[pl.ds(OUT_PER_W // 2, OUT_PER_W // 2)],
        out_hbm.at[pl.ds(wid * OUT_PER_W + OUT_PER_W // 2, OUT_PER_W // 2)],
        osem)
    pltpu.make_async_copy(out_v.at[pl.ds(0, OUT_PER_W // 2)],
                          out_hbm.at[pl.ds(0, OUT_PER_W // 2)], osem).wait()
    pltpu.make_async_copy(out_v.at[pl.ds(0, OUT_PER_W // 2)],
                          out_hbm.at[pl.ds(0, OUT_PER_W // 2)], osem).wait()


def kernel(indices, table):
    out = _embed_mean(indices.reshape(-1), table)
    return out.reshape(B, DIM)


# speedup vs baseline: 1.1744x; 1.0101x over previous
import functools

import jax
import jax.numpy as jnp
from jax import lax
from jax.experimental import pallas as pl
from jax.experimental.pallas import tpu as pltpu
from jax.experimental.pallas import tpu_sc as plsc

VOCAB = 1000000
DIM = 64
B = 4096
L = 200

NUM_CORES = 2
NUM_SUBCORES = 16
NW = NUM_CORES * NUM_SUBCORES
B_PER_W = B // NW
IDX_PER_W = B_PER_W * L
BPG = 2
GROWS = BPG * L
NGRP = B_PER_W // BPG
NBUF = 3
LANES = 16
NCH = DIM // LANES
OUT_PER_W = B_PER_W * DIM

_mesh = plsc.VectorSubcoreMesh(
    core_axis_name="c", subcore_axis_name="s",
    num_cores=NUM_CORES, num_subcores=NUM_SUBCORES)


@functools.partial(
    pl.kernel,
    out_type=jax.ShapeDtypeStruct((B * DIM,), jnp.float32),
    mesh=_mesh,
    scratch_types=[
        pltpu.VMEM((IDX_PER_W,), jnp.int32),
        [pltpu.VMEM((GROWS, DIM), jnp.float32) for _ in range(NBUF)],
        pltpu.VMEM((OUT_PER_W,), jnp.float32),
        pltpu.SemaphoreType.DMA,
        [pltpu.SemaphoreType.DMA for _ in range(NBUF)],
        pltpu.SemaphoreType.DMA,
    ],
    compiler_params=pltpu.CompilerParams(use_tc_tiling_on_sc=False),
)
def _embed_mean(idx_hbm, table_hbm, out_hbm, idx_v, bufs, out_v, isem,
                rsems, osem):
    wid = lax.axis_index("s") * NUM_CORES + lax.axis_index("c")
    ibase = wid * IDX_PER_W
    IHALF = IDX_PER_W // 2

    pltpu.async_copy(idx_hbm.at[pl.ds(ibase, IHALF)],
                     idx_v.at[pl.ds(0, IHALF)], isem)
    pltpu.async_copy(idx_hbm.at[pl.ds(ibase + IHALF, IHALF)],
                     idx_v.at[pl.ds(IHALF, IHALF)], isem)

    def gather(s, g):
        pltpu.async_copy(table_hbm.at[idx_v.at[pl.ds(g * GROWS, GROWS)]],
                         bufs[s], rsems[s])

    def wait_rows(s):
        pltpu.make_async_copy(table_hbm.at[pl.ds(0, GROWS)], bufs[s],
                              rsems[s]).wait()

    pltpu.make_async_copy(idx_hbm.at[pl.ds(0, IHALF)],
                          idx_v.at[pl.ds(0, IHALF)], isem).wait()
    for s in range(NBUF):
        gather(s, s)
    pltpu.make_async_copy(idx_hbm.at[pl.ds(0, IHALF)],
                          idx_v.at[pl.ds(0, IHALF)], isem).wait()

    scale = jnp.float32(1.0 / L)
    UNROLL = 8

    def accumulate(buf, g):
        for j in range(BPG):
            b = g * BPG + j

            def acc_body(i, carry, j=j):
                acc = list(carry)
                base = j * L + i * UNROLL
                for r in range(UNROLL):
                    for c in range(NCH):
                        k = c * 2 + (r % 2)
                        acc[k] = acc[k] + buf[base + r,
                                              pl.ds(c * LANES, LANES)]
                return tuple(acc)

            zero = jnp.zeros((LANES,), jnp.float32)
            acc = lax.fori_loop(0, L // UNROLL, acc_body, (zero,) * (2 * NCH))
            for c in range(NCH):
                out_v[pl.ds(b * DIM + c * LANES, LANES)] = (
                    acc[c * 2] + acc[c * 2 + 1]) * scale

    def body(i, carry):
        g0 = NBUF * i
        for off in range(NBUF):
            g = g0 + off
            wait_rows(off)
            accumulate(bufs[off], g)

            @pl.when(g + NBUF < NGRP)
            def _():
                gather(off, g + NBUF)
        return carry

    SPLIT_ITERS = 10
    SPLIT_EL = SPLIT_ITERS * NBUF * BPG * DIM
    lax.fori_loop(0, SPLIT_ITERS, body, 0)
    pltpu.async_copy(out_v.at[pl.ds(0, SPLIT_EL)],
                     out_hbm.at[pl.ds(wid * OUT_PER_W, SPLIT_EL)], osem)
    lax.fori_loop(SPLIT_ITERS, NGRP // NBUF, body, 0)

    REM = NGRP % NBUF
    for r in range(REM):
        wait_rows(r)
        accumulate(bufs[r], NGRP - REM + r)

    pltpu.async_copy(
        out_v.at[pl.ds(SPLIT_EL, OUT_PER_W - SPLIT_EL)],
        out_hbm.at[pl.ds(wid * OUT_PER_W + SPLIT_EL, OUT_PER_W - SPLIT_EL)],
        osem)
    pltpu.make_async_copy(out_v.at[pl.ds(0, SPLIT_EL)],
                          out_hbm.at[pl.ds(0, SPLIT_EL)], osem).wait()
    pltpu.make_async_copy(
        out_v.at[pl.ds(SPLIT_EL, OUT_PER_W - SPLIT_EL)],
        out_hbm.at[pl.ds(0, OUT_PER_W - SPLIT_EL)], osem).wait()


def kernel(indices, table):
    out = _embed_mean(indices.reshape(-1), table)
    return out.reshape(B, DIM)
